# baseline (speedup 1.0000x reference)
import jax
import jax.numpy as jnp
from jax.experimental import pallas as pl
from jax.experimental.pallas import tpu as pltpu

_B, _NQ, _NKV, _DIM, _H = 256, 8, 128, 512, 8
_HD = _DIM // _H
_BLK = _NKV // _NQ
_BB = 16
_R = _BB * _NKV
_QR = _BB * _NQ


def _fused_body(q_ref, k_ref, v_ref, wk_ref, wv_ref, wp_ref, bp_ref, out_ref,
                wkT_sc, wvT_sc, wpT_sc, e_sc, s_sc, st_sc, g_sc):
    scale = _HD ** -0.5

    @pl.when(pl.program_id(0) == 0)
    def _prologue():
        wkT_sc[...] = wk_ref[...].T.astype(jnp.bfloat16)
        wvT_sc[...] = wv_ref[...].T.astype(jnp.bfloat16)
        wpT_sc[...] = wp_ref[...].T.astype(jnp.bfloat16)
        r_i = jax.lax.broadcasted_iota(jnp.int32, (_R, _QR), 0)
        g_i = jax.lax.broadcasted_iota(jnp.int32, (_R, _QR), 1)
        e_sc[...] = (r_i // _BLK == g_i).astype(jnp.bfloat16)
        rT_i = jax.lax.broadcasted_iota(jnp.int32, (_QR, _R), 1)
        gT_i = jax.lax.broadcasted_iota(jnp.int32, (_QR, _R), 0)
        g_sc[...] = (rT_i // _BLK == gT_i).astype(jnp.bfloat16)
        c_i = jax.lax.broadcasted_iota(jnp.int32, (_DIM, _H), 0)
        h_i = jax.lax.broadcasted_iota(jnp.int32, (_DIM, _H), 1)
        s_sc[...] = (c_i // _HD == h_i).astype(jnp.bfloat16)
        cT_i = jax.lax.broadcasted_iota(jnp.int32, (_H, _DIM), 1)
        hT_i = jax.lax.broadcasted_iota(jnp.int32, (_H, _DIM), 0)
        st_sc[...] = (cT_i // _HD == hT_i).astype(jnp.bfloat16)

    kb = k_ref[...].reshape(_R, _DIM).astype(jnp.bfloat16)
    vb = v_ref[...].reshape(_R, _DIM).astype(jnp.bfloat16)
    qb = q_ref[...].reshape(_QR, _DIM).astype(jnp.bfloat16)
    kp = jnp.dot(kb, wkT_sc[...], preferred_element_type=jnp.float32)
    vp = jnp.dot(vb, wvT_sc[...], preferred_element_type=jnp.float32)
    qp = (jnp.dot(qb, wkT_sc[...], preferred_element_type=jnp.float32)
          * scale).astype(jnp.bfloat16)

    qe = jnp.dot(e_sc[...], qp, preferred_element_type=jnp.float32)
    prod = (kp * qe).astype(jnp.bfloat16)
    logits = jnp.dot(prod, s_sc[...],
                     preferred_element_type=jnp.float32)

    lg = logits.reshape(_QR, _BLK, _H)
    m = jnp.max(lg, axis=1, keepdims=True)
    e = jnp.exp(lg - m)
    s = jnp.sum(e, axis=1, keepdims=True)
    attn = (e / s).reshape(_R, _H)

    ae = jnp.dot(attn.astype(jnp.bfloat16), st_sc[...],
                 preferred_element_type=jnp.float32)
    w = (vp * ae).astype(jnp.bfloat16)
    x = jnp.dot(g_sc[...], w, preferred_element_type=jnp.float32)

    out = jnp.dot(x.astype(jnp.bfloat16), wpT_sc[...],
                  preferred_element_type=jnp.float32) + bp_ref[...]
    out_ref[...] = out.reshape(_BB, _NQ, _DIM)


def kernel(q, k, v, Wk, Wv, Wp, bp, attn_mask):
    del attn_mask
    bp2 = bp.reshape(1, _DIM)
    return pl.pallas_call(
        _fused_body,
        grid=(_B // _BB,),
        in_specs=[
            pl.BlockSpec((_BB, _NQ, _DIM), lambda i: (i, 0, 0)),
            pl.BlockSpec((_BB, _NKV, _DIM), lambda i: (i, 0, 0)),
            pl.BlockSpec((_BB, _NKV, _DIM), lambda i: (i, 0, 0)),
            pl.BlockSpec((_DIM, _DIM), lambda i: (0, 0)),
            pl.BlockSpec((_DIM, _DIM), lambda i: (0, 0)),
            pl.BlockSpec((_DIM, _DIM), lambda i: (0, 0)),
            pl.BlockSpec((1, _DIM), lambda i: (0, 0)),
        ],
        out_specs=pl.BlockSpec((_BB, _NQ, _DIM), lambda i: (i, 0, 0)),
        out_shape=jax.ShapeDtypeStruct((_B, _NQ, _DIM), jnp.float32),
        scratch_shapes=[
            pltpu.VMEM((_DIM, _DIM), jnp.bfloat16),
            pltpu.VMEM((_DIM, _DIM), jnp.bfloat16),
            pltpu.VMEM((_DIM, _DIM), jnp.bfloat16),
            pltpu.VMEM((_R, _QR), jnp.bfloat16),
            pltpu.VMEM((_DIM, _H), jnp.bfloat16),
            pltpu.VMEM((_H, _DIM), jnp.bfloat16),
            pltpu.VMEM((_QR, _R), jnp.bfloat16),
        ],
    )(q, k, v, Wk, Wv, Wp, bp2)
